# baseline (speedup 1.0000x reference)
import functools

import jax
import jax.numpy as jnp
from jax import lax
from jax.experimental import pallas as pl
from jax.experimental.pallas import tpu as pltpu
from jax.experimental.pallas import tpu_sc as plsc


_DEF = jax.lax.Precision.DEFAULT


def _passA_body(xyz1_ref, xyz2t_ref, i1_ref, i2_ref, i3_ref, w_ref, *, TN, S):
    b = pl.program_id(0)

    a = xyz1_ref[0]
    bt = xyz2t_ref[0]
    ax, ay, az = a[:, 0:1], a[:, 1:2], a[:, 2:3]
    bx, by, bz = bt[0:1, :], bt[1:2, :], bt[2:3, :]
    a_sq = ax * ax + ay * ay + az * az
    b_sq = bx * bx + by * by + bz * bz
    cross = jax.lax.dot(a, bt, precision=_DEF,
                        preferred_element_type=jnp.float32)
    d = a_sq + b_sq - 2.0 * cross

    inf = jnp.float32(jnp.inf)
    iota = jax.lax.broadcasted_iota(jnp.int32, (TN, S), 1)
    base = b * S

    m1 = jnp.min(d, axis=1, keepdims=True)
    e1 = d <= m1
    d2 = jnp.where(e1, inf, d)
    m2 = jnp.min(d2, axis=1, keepdims=True)
    e2 = d2 <= m2
    d3 = jnp.where(e2, inf, d2)
    m3 = jnp.min(d3, axis=1, keepdims=True)
    e3 = d3 <= m3

    i1_ref[0] = jnp.min(jnp.where(e1, iota, S), axis=1, keepdims=True) + base
    i2_ref[0] = jnp.min(jnp.where(e2, iota, S), axis=1, keepdims=True) + base
    i3_ref[0] = jnp.min(jnp.where(e3, iota, S), axis=1, keepdims=True) + base

    w1 = 1.0 / (m1 + 1e-8)
    w2 = 1.0 / (m2 + 1e-8)
    w3 = 1.0 / (m3 + 1e-8)
    wsum = w1 + w2 + w3
    ones16 = jnp.ones((1, 16), jnp.float32)
    w_ref[0] = jnp.concatenate(
        [(w1 / wsum) * ones16, (w2 / wsum) * ones16, (w3 / wsum) * ones16],
        axis=1)


def _make_sc_interp(BN, D2, CH):
    info = plsc.get_sparse_core_info()
    NC, NS = info.num_cores, info.num_subcores
    NW = NC * NS
    q_per_w = BN // NW
    n_chunks = q_per_w // CH
    mesh = plsc.VectorSubcoreMesh(core_axis_name="c", subcore_axis_name="s")

    @functools.partial(
        pl.kernel, mesh=mesh,
        out_type=jax.ShapeDtypeStruct((BN, D2), jnp.float32),
        scratch_types=[
            pltpu.VMEM((2, CH), jnp.int32),
            pltpu.VMEM((2, CH), jnp.int32),
            pltpu.VMEM((2, CH), jnp.int32),
            pltpu.VMEM((CH, 48), jnp.float32),
            pltpu.VMEM((2, CH, D2), jnp.float32),
            pltpu.VMEM((2, CH, D2), jnp.float32),
            pltpu.VMEM((2, CH, D2), jnp.float32),
            pltpu.VMEM((CH, D2), jnp.float32),
            pltpu.SemaphoreType.DMA,
            pltpu.SemaphoreType.DMA,
        ],
    )
    def sc_interp(table_hbm, i1_hbm, i2_hbm, i3_hbm, w_hbm, out_hbm,
                  i1v, i2v, i3v, wv, r1, r2, r3, outv, sem_a, sem_b):
        wid = lax.axis_index("s") * NC + lax.axis_index("c")
        wbase = wid * q_per_w
        sems = (sem_a, sem_b)

        def fire(ci, p):
            cbase = wbase + ci * CH
            pltpu.sync_copy(i1_hbm.at[pl.ds(cbase, CH)], i1v.at[p])
            pltpu.sync_copy(i2_hbm.at[pl.ds(cbase, CH)], i2v.at[p])
            pltpu.sync_copy(i3_hbm.at[pl.ds(cbase, CH)], i3v.at[p])
            return (pltpu.async_copy(table_hbm.at[i1v.at[p]], r1.at[p],
                                     sems[p]),
                    pltpu.async_copy(table_hbm.at[i2v.at[p]], r2.at[p],
                                     sems[p]),
                    pltpu.async_copy(table_hbm.at[i3v.at[p]], r3.at[p],
                                     sems[p]))

        def drain(p):
            pltpu.make_async_copy(table_hbm.at[i1v.at[p]], r1.at[p],
                                  sems[p]).wait()
            pltpu.make_async_copy(table_hbm.at[i2v.at[p]], r2.at[p],
                                  sems[p]).wait()
            pltpu.make_async_copy(table_hbm.at[i3v.at[p]], r3.at[p],
                                  sems[p]).wait()

        def combine(ci, p):
            cbase = wbase + ci * CH
            pltpu.sync_copy(w_hbm.at[pl.ds(cbase, CH)], wv)

            def q_body(qq, carry2):
                for u in range(4):
                    q = qq * 4 + u
                    wa = wv[q, pl.ds(0, 16)]
                    wb = wv[q, pl.ds(16, 16)]
                    wc = wv[q, pl.ds(32, 16)]
                    for j in range(D2 // 16):
                        sl = pl.ds(j * 16, 16)
                        outv[q, sl] = (wa * r1[p, q, sl]
                                       + wb * r2[p, q, sl]
                                       + wc * r3[p, q, sl])
                return carry2

            lax.fori_loop(0, CH // 4, q_body, 0)
            pltpu.sync_copy(outv, out_hbm.at[pl.ds(cbase, CH)])

        nhalf = n_chunks // 2
        fire(0, 0)

        def pair_body(k, carry):
            c0 = 2 * k
            fire(c0 + 1, 1)
            drain(0)
            combine(c0, 0)

            @pl.when(k + 1 < nhalf)
            def _():
                fire(c0 + 2, 0)

            drain(1)
            combine(c0 + 1, 1)
            return carry

        lax.fori_loop(0, nhalf, pair_body, 0)

    return sc_interp


def _passC_body(p1_ref, it_ref, w1t_ref, b1_ref, x1_ref, stats_ref):
    @pl.when(pl.program_id(0) == 0)
    def _():
        stats_ref[...] = jnp.zeros_like(stats_ref)

    D1 = p1_ref.shape[1]
    x1 = (jax.lax.dot(p1_ref[...], w1t_ref[:D1, :], precision=_DEF,
                      preferred_element_type=jnp.float32)
          + jax.lax.dot(it_ref[...], w1t_ref[D1:, :], precision=_DEF,
                        preferred_element_type=jnp.float32)
          + b1_ref[...])
    x1_ref[...] = x1
    x1sq = x1 * x1
    s = x1[0:8, :]
    sq = x1sq[0:8, :]
    for r in range(8, x1.shape[0], 8):
        s = s + x1[r:r + 8, :]
        sq = sq + x1sq[r:r + 8, :]
    stats_ref[0:8, :] = stats_ref[0:8, :] + s
    stats_ref[8:16, :] = stats_ref[8:16, :] + sq


def _pass2_body(x1_ref, sc_ref, sh_ref, w2t_ref, b2_ref, x2_ref, stats_ref):
    @pl.when(pl.program_id(0) == 0)
    def _():
        stats_ref[...] = jnp.zeros_like(stats_ref)

    h = jnp.maximum(x1_ref[...] * sc_ref[...] + sh_ref[...], 0.0)
    y = jax.lax.dot(h, w2t_ref[...], precision=_DEF,
                    preferred_element_type=jnp.float32) + b2_ref[...]
    x2_ref[...] = y
    ysq = y * y
    s = y[0:8, :]
    sq = ysq[0:8, :]
    for r in range(8, y.shape[0], 8):
        s = s + y[r:r + 8, :]
        sq = sq + ysq[r:r + 8, :]
    stats_ref[0:8, :] = stats_ref[0:8, :] + s
    stats_ref[8:16, :] = stats_ref[8:16, :] + sq


def _pass3_body(x2_ref, sc_ref, sh_ref, out_ref):
    out_ref[...] = jnp.maximum(x2_ref[...] * sc_ref[...] + sh_ref[...], 0.0)


def _affine(stats, gamma, beta, count):
    mean = jnp.sum(stats[0:8], axis=0) / count
    var = jnp.sum(stats[8:16], axis=0) / count - mean * mean
    scale = gamma / jnp.sqrt(var + 1e-5)
    shift = beta - mean * scale
    return scale[None, :], shift[None, :]


@jax.jit
def kernel(xyz1, xyz2, points1, points2, W1, b1, g1, be1, W2, b2, g2, be2):
    B, N, _ = xyz1.shape
    S = xyz2.shape[1]
    D1 = points1.shape[2]
    D2 = points2.shape[2]
    Cin = D1 + D2
    C = W1.shape[0]
    TN = 512

    xyz2t = jnp.transpose(xyz2, (0, 2, 1))
    w1t = jnp.transpose(W1)
    w2t = jnp.transpose(W2)
    b1r = b1[None, :]
    b2r = b2[None, :]

    BN = B * N
    sc_interp = _make_sc_interp(BN, D2, CH=64)
    TNC = 2048

    i1, i2, i3, wrep = pl.pallas_call(
        functools.partial(_passA_body, TN=TN, S=S),
        grid=(B, N // TN),
        in_specs=[
            pl.BlockSpec((1, TN, 3), lambda b, i: (b, i, 0)),
            pl.BlockSpec((1, 3, S), lambda b, i: (b, 0, 0)),
        ],
        out_specs=[
            pl.BlockSpec((1, TN, 1), lambda b, i: (b, i, 0)),
            pl.BlockSpec((1, TN, 1), lambda b, i: (b, i, 0)),
            pl.BlockSpec((1, TN, 1), lambda b, i: (b, i, 0)),
            pl.BlockSpec((1, TN, 48), lambda b, i: (b, i, 0)),
        ],
        out_shape=[
            jax.ShapeDtypeStruct((B, N, 1), jnp.int32),
            jax.ShapeDtypeStruct((B, N, 1), jnp.int32),
            jax.ShapeDtypeStruct((B, N, 1), jnp.int32),
            jax.ShapeDtypeStruct((B, N, 48), jnp.float32),
        ],
    )(xyz1, xyz2t)

    table = points2.reshape(B * S, D2)
    interp = sc_interp(table, i1.reshape(BN), i2.reshape(BN),
                       i3.reshape(BN), wrep.reshape(BN, 48))

    p1_2d = points1.reshape(BN, D1)
    x1, stats1 = pl.pallas_call(
        _passC_body,
        grid=(BN // TNC,),
        in_specs=[
            pl.BlockSpec((TNC, D1), lambda i: (i, 0)),
            pl.BlockSpec((TNC, D2), lambda i: (i, 0)),
            pl.BlockSpec((Cin, C), lambda i: (0, 0)),
            pl.BlockSpec((1, C), lambda i: (0, 0)),
        ],
        out_specs=[
            pl.BlockSpec((TNC, C), lambda i: (i, 0)),
            pl.BlockSpec((16, C), lambda i: (0, 0)),
        ],
        out_shape=[
            jax.ShapeDtypeStruct((BN, C), jnp.float32),
            jax.ShapeDtypeStruct((16, C), jnp.float32),
        ],
    )(p1_2d, interp, w1t, b1r)

    count = jnp.float32(BN)
    sc1, sh1 = _affine(stats1, g1, be1, count)

    TN2 = 2048
    x2, stats2 = pl.pallas_call(
        _pass2_body,
        grid=(BN // TN2,),
        in_specs=[
            pl.BlockSpec((TN2, C), lambda i: (i, 0)),
            pl.BlockSpec((1, C), lambda i: (0, 0)),
            pl.BlockSpec((1, C), lambda i: (0, 0)),
            pl.BlockSpec((C, C), lambda i: (0, 0)),
            pl.BlockSpec((1, C), lambda i: (0, 0)),
        ],
        out_specs=[
            pl.BlockSpec((TN2, C), lambda i: (i, 0)),
            pl.BlockSpec((16, C), lambda i: (0, 0)),
        ],
        out_shape=[
            jax.ShapeDtypeStruct((BN, C), jnp.float32),
            jax.ShapeDtypeStruct((16, C), jnp.float32),
        ],
    )(x1, sc1, sh1, w2t, b2r)

    sc2, sh2 = _affine(stats2, g2, be2, count)

    TN3 = 4096
    out = pl.pallas_call(
        _pass3_body,
        grid=(BN // TN3,),
        in_specs=[
            pl.BlockSpec((TN3, C), lambda i: (i, 0)),
            pl.BlockSpec((1, C), lambda i: (0, 0)),
            pl.BlockSpec((1, C), lambda i: (0, 0)),
        ],
        out_specs=pl.BlockSpec((TN3, C), lambda i: (i, 0)),
        out_shape=jax.ShapeDtypeStruct((BN, C), jnp.float32),
    )(x2, sc2, sh2)

    return out.reshape(B, N, C)


# speedup vs baseline: 1.1473x; 1.1473x over previous
import functools

import jax
import jax.numpy as jnp
from jax import lax
from jax.experimental import pallas as pl
from jax.experimental.pallas import tpu as pltpu
from jax.experimental.pallas import tpu_sc as plsc


_DEF = jax.lax.Precision.DEFAULT


def _passA_body(xyz1_ref, xyz2t_ref, i1_ref, i2_ref, i3_ref, w_ref, *, TN, S):
    b = pl.program_id(0)

    a = xyz1_ref[0]
    bt = xyz2t_ref[0]
    ax, ay, az = a[:, 0:1], a[:, 1:2], a[:, 2:3]
    bx, by, bz = bt[0:1, :], bt[1:2, :], bt[2:3, :]
    a_sq = ax * ax + ay * ay + az * az
    b_sq = bx * bx + by * by + bz * bz
    cross = jax.lax.dot(a, bt, precision=_DEF,
                        preferred_element_type=jnp.float32)
    d = a_sq + b_sq - 2.0 * cross

    inf = jnp.float32(jnp.inf)
    iota = jax.lax.broadcasted_iota(jnp.int32, (TN, S), 1)
    base = b * S

    m1 = jnp.min(d, axis=1, keepdims=True)
    e1 = d <= m1
    d2 = jnp.where(e1, inf, d)
    m2 = jnp.min(d2, axis=1, keepdims=True)
    e2 = d2 <= m2
    d3 = jnp.where(e2, inf, d2)
    m3 = jnp.min(d3, axis=1, keepdims=True)
    e3 = d3 <= m3

    i1_ref[0] = jnp.min(jnp.where(e1, iota, S), axis=1, keepdims=True) + base
    i2_ref[0] = jnp.min(jnp.where(e2, iota, S), axis=1, keepdims=True) + base
    i3_ref[0] = jnp.min(jnp.where(e3, iota, S), axis=1, keepdims=True) + base

    w1 = 1.0 / (m1 + 1e-8)
    w2 = 1.0 / (m2 + 1e-8)
    w3 = 1.0 / (m3 + 1e-8)
    wsum = w1 + w2 + w3
    ones16 = jnp.ones((1, 16), jnp.float32)
    w_ref[0] = jnp.concatenate(
        [(w1 / wsum) * ones16, (w2 / wsum) * ones16, (w3 / wsum) * ones16],
        axis=1)


def _make_sc_interp(BN, D2, CH):
    info = plsc.get_sparse_core_info()
    NC, NS = info.num_cores, info.num_subcores
    NW = NC * NS
    q_per_w = BN // NW
    n_chunks = q_per_w // CH
    mesh = plsc.VectorSubcoreMesh(core_axis_name="c", subcore_axis_name="s")

    @functools.partial(
        pl.kernel, mesh=mesh,
        out_type=jax.ShapeDtypeStruct((BN, D2), jnp.float32),
        scratch_types=[
            pltpu.VMEM((2, CH), jnp.int32),
            pltpu.VMEM((2, CH), jnp.int32),
            pltpu.VMEM((2, CH), jnp.int32),
            pltpu.VMEM((CH, 48), jnp.float32),
            pltpu.VMEM((2, CH, D2), jnp.float32),
            pltpu.VMEM((2, CH, D2), jnp.float32),
            pltpu.VMEM((2, CH, D2), jnp.float32),
            pltpu.VMEM((CH, D2), jnp.float32),
            pltpu.SemaphoreType.DMA,
            pltpu.SemaphoreType.DMA,
        ],
    )
    def sc_interp(table_hbm, i1_hbm, i2_hbm, i3_hbm, w_hbm, out_hbm,
                  i1v, i2v, i3v, wv, r1, r2, r3, outv, sem_a, sem_b):
        wid = lax.axis_index("s") * NC + lax.axis_index("c")
        wbase = wid * q_per_w
        sems = (sem_a, sem_b)

        def fire(ci, p):
            cbase = wbase + ci * CH
            pltpu.sync_copy(i1_hbm.at[pl.ds(cbase, CH)], i1v.at[p])
            pltpu.sync_copy(i2_hbm.at[pl.ds(cbase, CH)], i2v.at[p])
            pltpu.sync_copy(i3_hbm.at[pl.ds(cbase, CH)], i3v.at[p])
            return (pltpu.async_copy(table_hbm.at[i1v.at[p]], r1.at[p],
                                     sems[p]),
                    pltpu.async_copy(table_hbm.at[i2v.at[p]], r2.at[p],
                                     sems[p]),
                    pltpu.async_copy(table_hbm.at[i3v.at[p]], r3.at[p],
                                     sems[p]))

        pend = fire(0, 0)
        for ci in range(n_chunks):
            p = ci % 2
            if ci + 1 < n_chunks:
                nxt = fire(ci + 1, 1 - p)
            for c in pend:
                c.wait()
            cbase = wbase + ci * CH
            pltpu.sync_copy(w_hbm.at[pl.ds(cbase, CH)], wv)

            def q_body(qq, carry2, _p=p):
                for u in range(2):
                    q = qq * 2 + u
                    wa = wv[q, pl.ds(0, 16)]
                    wb = wv[q, pl.ds(16, 16)]
                    wc = wv[q, pl.ds(32, 16)]
                    for j in range(D2 // 16):
                        sl = pl.ds(j * 16, 16)
                        outv[q, sl] = (wa * r1[_p, q, sl]
                                       + wb * r2[_p, q, sl]
                                       + wc * r3[_p, q, sl])
                return carry2

            lax.fori_loop(0, CH // 2, q_body, 0)
            pltpu.sync_copy(outv, out_hbm.at[pl.ds(cbase, CH)])
            if ci + 1 < n_chunks:
                pend = nxt

    return sc_interp


def _passC_body(p1_ref, it_ref, w1t_ref, b1_ref, x1_ref, stats_ref):
    @pl.when(pl.program_id(0) == 0)
    def _():
        stats_ref[...] = jnp.zeros_like(stats_ref)

    D1 = p1_ref.shape[1]
    x1 = (jax.lax.dot(p1_ref[...], w1t_ref[:D1, :], precision=_DEF,
                      preferred_element_type=jnp.float32)
          + jax.lax.dot(it_ref[...], w1t_ref[D1:, :], precision=_DEF,
                        preferred_element_type=jnp.float32)
          + b1_ref[...])
    x1_ref[...] = x1
    x1sq = x1 * x1
    s = x1[0:8, :]
    sq = x1sq[0:8, :]
    for r in range(8, x1.shape[0], 8):
        s = s + x1[r:r + 8, :]
        sq = sq + x1sq[r:r + 8, :]
    stats_ref[0:8, :] = stats_ref[0:8, :] + s
    stats_ref[8:16, :] = stats_ref[8:16, :] + sq


def _pass2_body(x1_ref, sc_ref, sh_ref, w2t_ref, b2_ref, x2_ref, stats_ref):
    @pl.when(pl.program_id(0) == 0)
    def _():
        stats_ref[...] = jnp.zeros_like(stats_ref)

    h = jnp.maximum(x1_ref[...] * sc_ref[...] + sh_ref[...], 0.0)
    y = jax.lax.dot(h, w2t_ref[...], precision=_DEF,
                    preferred_element_type=jnp.float32) + b2_ref[...]
    x2_ref[...] = y
    ysq = y * y
    s = y[0:8, :]
    sq = ysq[0:8, :]
    for r in range(8, y.shape[0], 8):
        s = s + y[r:r + 8, :]
        sq = sq + ysq[r:r + 8, :]
    stats_ref[0:8, :] = stats_ref[0:8, :] + s
    stats_ref[8:16, :] = stats_ref[8:16, :] + sq


def _pass3_body(x2_ref, sc_ref, sh_ref, out_ref):
    out_ref[...] = jnp.maximum(x2_ref[...] * sc_ref[...] + sh_ref[...], 0.0)


def _affine(stats, gamma, beta, count):
    mean = jnp.sum(stats[0:8], axis=0) / count
    var = jnp.sum(stats[8:16], axis=0) / count - mean * mean
    scale = gamma / jnp.sqrt(var + 1e-5)
    shift = beta - mean * scale
    return scale[None, :], shift[None, :]


@jax.jit
def kernel(xyz1, xyz2, points1, points2, W1, b1, g1, be1, W2, b2, g2, be2):
    B, N, _ = xyz1.shape
    S = xyz2.shape[1]
    D1 = points1.shape[2]
    D2 = points2.shape[2]
    Cin = D1 + D2
    C = W1.shape[0]
    TN = 512

    xyz2t = jnp.transpose(xyz2, (0, 2, 1))
    w1t = jnp.transpose(W1)
    w2t = jnp.transpose(W2)
    b1r = b1[None, :]
    b2r = b2[None, :]

    BN = B * N
    NH = 2
    B2 = B // NH
    BN2 = B2 * N
    sc_interp = _make_sc_interp(BN2, D2, CH=64)
    TNC = 2048

    x1_halves = []
    stats1 = jnp.zeros((16, C), jnp.float32)
    for h in range(NH):
        xyz1h = xyz1[h * B2:(h + 1) * B2]
        xyz2th = xyz2t[h * B2:(h + 1) * B2]
        i1, i2, i3, wrep = pl.pallas_call(
            functools.partial(_passA_body, TN=TN, S=S),
            grid=(B2, N // TN),
            in_specs=[
                pl.BlockSpec((1, TN, 3), lambda b, i: (b, i, 0)),
                pl.BlockSpec((1, 3, S), lambda b, i: (b, 0, 0)),
            ],
            out_specs=[
                pl.BlockSpec((1, TN, 1), lambda b, i: (b, i, 0)),
                pl.BlockSpec((1, TN, 1), lambda b, i: (b, i, 0)),
                pl.BlockSpec((1, TN, 1), lambda b, i: (b, i, 0)),
                pl.BlockSpec((1, TN, 48), lambda b, i: (b, i, 0)),
            ],
            out_shape=[
                jax.ShapeDtypeStruct((B2, N, 1), jnp.int32),
                jax.ShapeDtypeStruct((B2, N, 1), jnp.int32),
                jax.ShapeDtypeStruct((B2, N, 1), jnp.int32),
                jax.ShapeDtypeStruct((B2, N, 48), jnp.float32),
            ],
        )(xyz1h, xyz2th)

        table = points2[h * B2:(h + 1) * B2].reshape(B2 * S, D2)
        interp = sc_interp(table, i1.reshape(BN2), i2.reshape(BN2),
                           i3.reshape(BN2), wrep.reshape(BN2, 48))

        p1_2d = points1[h * B2:(h + 1) * B2].reshape(BN2, D1)
        x1h, stats1h = pl.pallas_call(
            _passC_body,
            grid=(BN2 // TNC,),
            in_specs=[
                pl.BlockSpec((TNC, D1), lambda i: (i, 0)),
                pl.BlockSpec((TNC, D2), lambda i: (i, 0)),
                pl.BlockSpec((Cin, C), lambda i: (0, 0)),
                pl.BlockSpec((1, C), lambda i: (0, 0)),
            ],
            out_specs=[
                pl.BlockSpec((TNC, C), lambda i: (i, 0)),
                pl.BlockSpec((16, C), lambda i: (0, 0)),
            ],
            out_shape=[
                jax.ShapeDtypeStruct((BN2, C), jnp.float32),
                jax.ShapeDtypeStruct((16, C), jnp.float32),
            ],
        )(p1_2d, interp, w1t, b1r)
        x1_halves.append(x1h)
        stats1 = stats1 + stats1h

    x1 = jnp.concatenate(x1_halves, axis=0)

    count = jnp.float32(BN)
    sc1, sh1 = _affine(stats1, g1, be1, count)

    TN2 = 2048
    x2, stats2 = pl.pallas_call(
        _pass2_body,
        grid=(BN // TN2,),
        in_specs=[
            pl.BlockSpec((TN2, C), lambda i: (i, 0)),
            pl.BlockSpec((1, C), lambda i: (0, 0)),
            pl.BlockSpec((1, C), lambda i: (0, 0)),
            pl.BlockSpec((C, C), lambda i: (0, 0)),
            pl.BlockSpec((1, C), lambda i: (0, 0)),
        ],
        out_specs=[
            pl.BlockSpec((TN2, C), lambda i: (i, 0)),
            pl.BlockSpec((16, C), lambda i: (0, 0)),
        ],
        out_shape=[
            jax.ShapeDtypeStruct((BN, C), jnp.float32),
            jax.ShapeDtypeStruct((16, C), jnp.float32),
        ],
    )(x1, sc1, sh1, w2t, b2r)

    sc2, sh2 = _affine(stats2, g2, be2, count)

    TN3 = 4096
    out = pl.pallas_call(
        _pass3_body,
        grid=(BN // TN3,),
        in_specs=[
            pl.BlockSpec((TN3, C), lambda i: (i, 0)),
            pl.BlockSpec((1, C), lambda i: (0, 0)),
            pl.BlockSpec((1, C), lambda i: (0, 0)),
        ],
        out_specs=pl.BlockSpec((TN3, C), lambda i: (i, 0)),
        out_shape=jax.ShapeDtypeStruct((BN, C), jnp.float32),
    )(x2, sc2, sh2)

    return out.reshape(B, N, C)
